# baseline (speedup 1.0000x reference)
import functools

import jax
import jax.numpy as jnp
from jax import lax
from jax.experimental import pallas as pl
from jax.experimental.pallas import tpu as pltpu
from jax.experimental.pallas import tpu_sc as plsc


@functools.cache
def _make_transpose(V, D):
    BC = 512
    grid = (V + BC - 1) // BC

    def tr_kernel(in_ref, out_ref):
        out_ref[:, pl.ds(0, D)] = in_ref[...].T

    return pl.pallas_call(
        tr_kernel,
        grid=(grid,),
        in_specs=[pl.BlockSpec((D, BC), lambda i: (0, i))],
        out_specs=pl.BlockSpec((BC, 2 * D), lambda i: (i, 0)),
        out_shape=jax.ShapeDtypeStruct((V, 2 * D), jnp.float32),
    )


@functools.cache
def _make_gather(V, D, B, L):
    info = plsc.get_sparse_core_info()
    NC, NS = info.num_cores, info.num_subcores
    NW = NC * NS
    assert B % NW == 0
    CB = B // NW
    W = 2 * D

    mesh = plsc.VectorSubcoreMesh(core_axis_name="c", subcore_axis_name="s")

    @functools.partial(
        pl.kernel,
        mesh=mesh,
        out_type=jax.ShapeDtypeStruct((L, B, W), jnp.float32),
        compiler_params=pltpu.CompilerParams(use_tc_tiling_on_sc=True),
        scratch_types=[
            pltpu.VMEM((L, CB), jnp.int32),
            pltpu.VMEM((2, CB, W), jnp.float32),
            pltpu.SemaphoreType.DMA,
            pltpu.SemaphoreType.DMA,
        ],
    )
    def gather_kernel(table_hbm, idxt_hbm, out_hbm, idx_v, rows_v, gsem, wsem):
        wid = lax.axis_index("s") * NC + lax.axis_index("c")
        c0 = wid * CB
        pltpu.sync_copy(idxt_hbm.at[:, pl.ds(c0, CB)], idx_v)

        def gather(l, slot):
            return pltpu.async_copy(
                table_hbm.at[idx_v.at[l]], rows_v.at[slot], gsem
            )

        def gather_wait(slot):
            pltpu.make_async_copy(
                table_hbm.at[idx_v.at[0]], rows_v.at[slot], gsem
            ).wait()

        def write(l, slot):
            return pltpu.async_copy(
                rows_v.at[slot], out_hbm.at[l, pl.ds(c0, CB)], wsem
            )

        def write_wait(slot):
            pltpu.make_async_copy(
                rows_v.at[slot], out_hbm.at[0, pl.ds(c0, CB)], wsem
            ).wait()

        gather(0, 0)

        def body(l, carry):
            cur = lax.rem(l, 2)

            gather_wait(cur)

            @pl.when(l >= 1)
            def _():
                write_wait(1 - cur)

            @pl.when(l + 1 < L)
            def _():
                gather(l + 1, 1 - cur)

            write(l, cur)
            return carry

        lax.fori_loop(0, L, body, 0)
        write_wait((L - 1) % 2)

    return gather_kernel


def kernel(word_table, indices):
    B, L = indices.shape
    V, D = word_table.shape
    wt_wide = _make_transpose(V, D)(word_table.T)
    out_t = _make_gather(V, D, B, L)(wt_wide, indices.T)
    return out_t.transpose(1, 0, 2)[..., :D]


# speedup vs baseline: 2.7787x; 2.7787x over previous
import functools

import jax
import jax.numpy as jnp
from jax import lax
from jax.experimental import pallas as pl
from jax.experimental.pallas import tpu as pltpu
from jax.experimental.pallas import tpu_sc as plsc


@functools.cache
def _make_transpose(V, D):
    BC = 4096
    grid = (V + BC - 1) // BC

    def tr_kernel(in_ref, out_ref):
        out_ref[:, pl.ds(0, D)] = in_ref[...].T

    return pl.pallas_call(
        tr_kernel,
        grid=(grid,),
        in_specs=[pl.BlockSpec((D, BC), lambda i: (0, i))],
        out_specs=pl.BlockSpec((BC, 2 * D), lambda i: (i, 0)),
        out_shape=jax.ShapeDtypeStruct((V, 2 * D), jnp.float32),
    )


@functools.cache
def _make_gather(V, D, B, L):
    info = plsc.get_sparse_core_info()
    NC, NS = info.num_cores, info.num_subcores
    NW = NC * NS
    assert B % NW == 0
    CB = B // NW
    W = 2 * D

    mesh = plsc.VectorSubcoreMesh(core_axis_name="c", subcore_axis_name="s")

    @functools.partial(
        pl.kernel,
        mesh=mesh,
        out_type=jax.ShapeDtypeStruct((L, B, W), jnp.float32),
        compiler_params=pltpu.CompilerParams(use_tc_tiling_on_sc=True),
        scratch_types=[
            pltpu.VMEM((L, CB), jnp.int32),
            pltpu.VMEM((2, CB, W), jnp.float32),
            pltpu.SemaphoreType.DMA,
            pltpu.SemaphoreType.DMA,
        ],
    )
    def gather_kernel(table_hbm, idxt_hbm, out_hbm, idx_v, rows_v, gsem, wsem):
        wid = lax.axis_index("s") * NC + lax.axis_index("c")
        c0 = wid * CB
        pltpu.sync_copy(idxt_hbm.at[:, pl.ds(c0, CB)], idx_v)

        def gather(l, slot):
            return pltpu.async_copy(
                table_hbm.at[idx_v.at[l]], rows_v.at[slot], gsem
            )

        def gather_wait(slot):
            pltpu.make_async_copy(
                table_hbm.at[idx_v.at[0]], rows_v.at[slot], gsem
            ).wait()

        def write(l, slot):
            return pltpu.async_copy(
                rows_v.at[slot], out_hbm.at[l, pl.ds(c0, CB)], wsem
            )

        def write_wait(slot):
            pltpu.make_async_copy(
                rows_v.at[slot], out_hbm.at[0, pl.ds(c0, CB)], wsem
            ).wait()

        gather(0, 0)

        def body(l, carry):
            cur = lax.rem(l, 2)

            gather_wait(cur)

            @pl.when(l >= 1)
            def _():
                write_wait(1 - cur)

            @pl.when(l + 1 < L)
            def _():
                gather(l + 1, 1 - cur)

            write(l, cur)
            return carry

        lax.fori_loop(0, L, body, 0)
        write_wait((L - 1) % 2)

    return gather_kernel


def kernel(word_table, indices):
    B, L = indices.shape
    V, D = word_table.shape
    wt_wide = _make_transpose(V, D)(word_table.T)
    out_t = _make_gather(V, D, B, L)(wt_wide, indices.T)
    return out_t.transpose(1, 0, 2)[..., :D]


# speedup vs baseline: 3.2669x; 1.1757x over previous
import functools

import jax
import jax.numpy as jnp
from jax import lax
from jax.experimental import pallas as pl
from jax.experimental.pallas import tpu as pltpu
from jax.experimental.pallas import tpu_sc as plsc


@functools.cache
def _make_transpose(V, D):
    BC = 8192
    grid = (V + BC - 1) // BC

    def tr_kernel(in_ref, out_ref):
        out_ref[:, pl.ds(0, D)] = in_ref[...].T

    return pl.pallas_call(
        tr_kernel,
        grid=(grid,),
        in_specs=[pl.BlockSpec((D, BC), lambda i: (0, i))],
        out_specs=pl.BlockSpec((BC, 2 * D), lambda i: (i, 0)),
        out_shape=jax.ShapeDtypeStruct((V, 2 * D), jnp.float32),
    )


@functools.cache
def _make_gather(V, D, B, L):
    info = plsc.get_sparse_core_info()
    NC, NS = info.num_cores, info.num_subcores
    NW = NC * NS
    assert B % NW == 0
    CB = B // NW
    W = 2 * D

    mesh = plsc.VectorSubcoreMesh(core_axis_name="c", subcore_axis_name="s")

    @functools.partial(
        pl.kernel,
        mesh=mesh,
        out_type=jax.ShapeDtypeStruct((L, B, W), jnp.float32),
        compiler_params=pltpu.CompilerParams(use_tc_tiling_on_sc=True),
        scratch_types=[
            pltpu.VMEM((L, CB), jnp.int32),
            pltpu.VMEM((2, CB, W), jnp.float32),
            pltpu.SemaphoreType.DMA,
            pltpu.SemaphoreType.DMA,
        ],
    )
    def gather_kernel(table_hbm, idxt_hbm, out_hbm, idx_v, rows_v, gsem, wsem):
        wid = lax.axis_index("s") * NC + lax.axis_index("c")
        c0 = wid * CB
        pltpu.sync_copy(idxt_hbm.at[:, pl.ds(c0, CB)], idx_v)

        def gather(l, slot):
            return pltpu.async_copy(
                table_hbm.at[idx_v.at[l]], rows_v.at[slot], gsem
            )

        def gather_wait(slot):
            pltpu.make_async_copy(
                table_hbm.at[idx_v.at[0]], rows_v.at[slot], gsem
            ).wait()

        def write(l, slot):
            return pltpu.async_copy(
                rows_v.at[slot], out_hbm.at[l, pl.ds(c0, CB)], wsem
            )

        def write_wait(slot):
            pltpu.make_async_copy(
                rows_v.at[slot], out_hbm.at[0, pl.ds(c0, CB)], wsem
            ).wait()

        gather(0, 0)

        def body(l, carry):
            cur = lax.rem(l, 2)

            gather_wait(cur)

            @pl.when(l >= 1)
            def _():
                write_wait(1 - cur)

            @pl.when(l + 1 < L)
            def _():
                gather(l + 1, 1 - cur)

            write(l, cur)
            return carry

        lax.fori_loop(0, L, body, 0)
        write_wait((L - 1) % 2)

    return gather_kernel


def kernel(word_table, indices):
    B, L = indices.shape
    V, D = word_table.shape
    wt_wide = _make_transpose(V, D)(word_table.T)
    out_t = _make_gather(V, D, B, L)(wt_wide, indices.T)
    return out_t.transpose(1, 0, 2)[..., :D]


# speedup vs baseline: 3.4312x; 1.0503x over previous
import functools

import jax
import jax.numpy as jnp
from jax import lax
from jax.experimental import pallas as pl
from jax.experimental.pallas import tpu as pltpu
from jax.experimental.pallas import tpu_sc as plsc


@functools.cache
def _make_transpose(V, D):
    BC = 16384
    grid = (V + BC - 1) // BC

    def tr_kernel(in_ref, out_ref):
        out_ref[:, pl.ds(0, D)] = in_ref[...].T

    return pl.pallas_call(
        tr_kernel,
        grid=(grid,),
        in_specs=[pl.BlockSpec((D, BC), lambda i: (0, i))],
        out_specs=pl.BlockSpec((BC, 2 * D), lambda i: (i, 0)),
        out_shape=jax.ShapeDtypeStruct((V, 2 * D), jnp.float32),
    )


@functools.cache
def _make_gather(V, D, B, L):
    info = plsc.get_sparse_core_info()
    NC, NS = info.num_cores, info.num_subcores
    NW = NC * NS
    assert B % NW == 0
    CB = B // NW
    W = 2 * D

    mesh = plsc.VectorSubcoreMesh(core_axis_name="c", subcore_axis_name="s")

    @functools.partial(
        pl.kernel,
        mesh=mesh,
        out_type=jax.ShapeDtypeStruct((L, B, W), jnp.float32),
        compiler_params=pltpu.CompilerParams(use_tc_tiling_on_sc=True),
        scratch_types=[
            pltpu.VMEM((L, CB), jnp.int32),
            pltpu.VMEM((2, CB, W), jnp.float32),
            pltpu.SemaphoreType.DMA,
            pltpu.SemaphoreType.DMA,
        ],
    )
    def gather_kernel(table_hbm, idxt_hbm, out_hbm, idx_v, rows_v, gsem, wsem):
        wid = lax.axis_index("s") * NC + lax.axis_index("c")
        c0 = wid * CB
        pltpu.sync_copy(idxt_hbm.at[:, pl.ds(c0, CB)], idx_v)

        def gather(l, slot):
            return pltpu.async_copy(
                table_hbm.at[idx_v.at[l]], rows_v.at[slot], gsem
            )

        def gather_wait(slot):
            pltpu.make_async_copy(
                table_hbm.at[idx_v.at[0]], rows_v.at[slot], gsem
            ).wait()

        def write(l, slot):
            return pltpu.async_copy(
                rows_v.at[slot], out_hbm.at[l, pl.ds(c0, CB)], wsem
            )

        def write_wait(slot):
            pltpu.make_async_copy(
                rows_v.at[slot], out_hbm.at[0, pl.ds(c0, CB)], wsem
            ).wait()

        gather(0, 0)

        def body(l, carry):
            cur = lax.rem(l, 2)

            gather_wait(cur)

            @pl.when(l >= 1)
            def _():
                write_wait(1 - cur)

            @pl.when(l + 1 < L)
            def _():
                gather(l + 1, 1 - cur)

            write(l, cur)
            return carry

        lax.fori_loop(0, L, body, 0)
        write_wait((L - 1) % 2)

    return gather_kernel


def kernel(word_table, indices):
    B, L = indices.shape
    V, D = word_table.shape
    wt_wide = _make_transpose(V, D)(word_table.T)
    out_t = _make_gather(V, D, B, L)(wt_wide, indices.T)
    return out_t.transpose(1, 0, 2)[..., :D]


# speedup vs baseline: 3.4839x; 1.0153x over previous
import functools

import jax
import jax.numpy as jnp
from jax import lax
from jax.experimental import pallas as pl
from jax.experimental.pallas import tpu as pltpu
from jax.experimental.pallas import tpu_sc as plsc


@functools.cache
def _make_transpose(V, D):
    BC = 32768
    grid = (V + BC - 1) // BC

    def tr_kernel(in_ref, out_ref):
        out_ref[:, pl.ds(0, D)] = in_ref[...].T

    return pl.pallas_call(
        tr_kernel,
        grid=(grid,),
        in_specs=[pl.BlockSpec((D, BC), lambda i: (0, i))],
        out_specs=pl.BlockSpec((BC, 2 * D), lambda i: (i, 0)),
        out_shape=jax.ShapeDtypeStruct((V, 2 * D), jnp.float32),
    )


@functools.cache
def _make_gather(V, D, B, L):
    info = plsc.get_sparse_core_info()
    NC, NS = info.num_cores, info.num_subcores
    NW = NC * NS
    assert B % NW == 0
    CB = B // NW
    W = 2 * D

    mesh = plsc.VectorSubcoreMesh(core_axis_name="c", subcore_axis_name="s")

    @functools.partial(
        pl.kernel,
        mesh=mesh,
        out_type=jax.ShapeDtypeStruct((L, B, W), jnp.float32),
        compiler_params=pltpu.CompilerParams(use_tc_tiling_on_sc=True),
        scratch_types=[
            pltpu.VMEM((L, CB), jnp.int32),
            pltpu.VMEM((2, CB, W), jnp.float32),
            pltpu.SemaphoreType.DMA,
            pltpu.SemaphoreType.DMA,
        ],
    )
    def gather_kernel(table_hbm, idxt_hbm, out_hbm, idx_v, rows_v, gsem, wsem):
        wid = lax.axis_index("s") * NC + lax.axis_index("c")
        c0 = wid * CB
        pltpu.sync_copy(idxt_hbm.at[:, pl.ds(c0, CB)], idx_v)

        def gather(l, slot):
            return pltpu.async_copy(
                table_hbm.at[idx_v.at[l]], rows_v.at[slot], gsem
            )

        def gather_wait(slot):
            pltpu.make_async_copy(
                table_hbm.at[idx_v.at[0]], rows_v.at[slot], gsem
            ).wait()

        def write(l, slot):
            return pltpu.async_copy(
                rows_v.at[slot], out_hbm.at[l, pl.ds(c0, CB)], wsem
            )

        def write_wait(slot):
            pltpu.make_async_copy(
                rows_v.at[slot], out_hbm.at[0, pl.ds(c0, CB)], wsem
            ).wait()

        gather(0, 0)

        def body(l, carry):
            cur = lax.rem(l, 2)

            gather_wait(cur)

            @pl.when(l >= 1)
            def _():
                write_wait(1 - cur)

            @pl.when(l + 1 < L)
            def _():
                gather(l + 1, 1 - cur)

            write(l, cur)
            return carry

        lax.fori_loop(0, L, body, 0)
        write_wait((L - 1) % 2)

    return gather_kernel


def kernel(word_table, indices):
    B, L = indices.shape
    V, D = word_table.shape
    wt_wide = _make_transpose(V, D)(word_table.T)
    out_t = _make_gather(V, D, B, L)(wt_wide, indices.T)
    return out_t.transpose(1, 0, 2)[..., :D]
